# Initial kernel scaffold; baseline (speedup 1.0000x reference)
import jax
import jax.numpy as jnp
from jax.experimental import pallas as pl


def kernel(x, V0, g0, b0, V1, g1, b1, V2, g2, b2, V3, g3, b3, fcV, fcg, fcb):
    raise NotImplementedError("write your pallas kernel here")


# speedup vs baseline: 732.8164x; 732.8164x over previous
import jax
import jax.numpy as jnp
from jax import lax
from jax.experimental import pallas as pl

_N = 16
_J = 128
_SEG = 1024
_K = 20

_DN = (((1,), (1,)), ((), ()))


def _fold_body(s_ref, v0, g0, b0, v1, g1, b1, v2, g2, b2, v3, g3, b3,
               fcv, fcg, fcb, out_ref):
    f32 = jnp.float32
    hp = lax.Precision.HIGHEST

    def wn(v, g, axis):
        n = jnp.sqrt(jnp.sum(v * v, axis=axis, keepdims=True) + 1e-12)
        return v * (g / n)

    W0 = wn(v0[...], g0[...], 0)
    W1 = wn(v1[...], g1[...], 0)
    W2 = wn(v2[...], g2[...], 0)
    W3 = wn(v3[...], g3[...], 0)

    a1 = jnp.dot(W0, W1, precision=hp)
    c1 = 4.0 * jnp.dot(b0[...], W1, precision=hp) + b1[...]
    a2 = a1 + jnp.dot(a1, W2, precision=hp)
    c2 = c1 + jnp.dot(c1, W2, precision=hp) + b2[...]
    a3 = jnp.dot(a2, W3, precision=hp)
    c3 = 128.0 * jnp.dot(c2, W3, precision=hp) + b3[...]

    fcw = wn(fcv[...], fcg[...], 1)
    fcw3 = fcw.reshape(_K, _J, 128)
    Mt = jnp.sum(fcw3 * a3[0][None, None, :], axis=-1)
    Mc = jnp.sum(fcw3 * c3[0][None, None, :], axis=-1)

    s = s_ref[...].astype(f32)
    ones = jnp.ones((1, _J), f32)
    out = lax.dot_general(s, Mt, _DN, precision=hp)
    out += lax.dot_general(ones, Mc, _DN, precision=hp)
    out_ref[...] = out + fcb[...]


def _pool_body(x_ref, s_ref):
    s_ref[...] = jnp.sum(x_ref[...], axis=-1)


def kernel(x, V0, g0, b0, V1, g1, b1, V2, g2, b2, V3, g3, b3, fcV, fcg, fcb):
    x3 = x.reshape(_N, _J, _SEG)

    s = pl.pallas_call(
        _pool_body,
        out_shape=jax.ShapeDtypeStruct((_N, _J), jnp.float32),
    )(x3)

    args = (
        s,
        V0, g0.reshape(1, -1), b0.reshape(1, -1),
        V1, g1.reshape(1, -1), b1.reshape(1, -1),
        V2, g2.reshape(1, -1), b2.reshape(1, -1),
        V3, g3.reshape(1, -1), b3.reshape(1, -1),
        fcV, fcg.reshape(-1, 1), fcb.reshape(1, -1),
    )
    return pl.pallas_call(
        _fold_body,
        out_shape=jax.ShapeDtypeStruct((_N, _K), jnp.float32),
    )(*args)
